# baseline (device time: 17401 ns/iter reference)
import jax
import jax.numpy as jnp
from jax import lax
from jax.experimental import pallas as pl
from jax.experimental.pallas import tpu as pltpu

N_DEV = 16


def kernel(x):
    m_per, n_per = x.shape

    def body(x_ref, out_ref, local_ref, stats_ref, send_sems, recv_sems,
             credit_sem):
        my = lax.axis_index("i")

        barrier_sem = pltpu.get_barrier_semaphore()
        for j in range(N_DEV):
            @pl.when(j != my)
            def _(j=j):
                pl.semaphore_signal(
                    barrier_sem, inc=1,
                    device_id=(j,), device_id_type=pl.DeviceIdType.MESH,
                )

        xv = x_ref[:, :]
        e = jnp.exp(xv)
        s = jnp.sum(e, axis=1, keepdims=True)

        packed = jnp.transpose(s)
        local_ref[:, :] = packed
        stats_ref[pl.ds(my, 1), :, :] = packed[None, :, :]

        for j in range(N_DEV):
            @pl.when(j != my)
            def _(j=j):
                rdma = pltpu.make_async_remote_copy(
                    src_ref=local_ref,
                    dst_ref=stats_ref.at[my],
                    send_sem=send_sems.at[j],
                    recv_sem=recv_sems.at[my],
                    device_id=(j,),
                    device_id_type=pl.DeviceIdType.MESH,
                )
                rdma.start()

        for j in range(N_DEV):
            @pl.when(j != my)
            def _(j=j):
                recv = pltpu.make_async_remote_copy(
                    src_ref=local_ref,
                    dst_ref=stats_ref.at[j],
                    send_sem=send_sems.at[j],
                    recv_sem=recv_sems.at[j],
                    device_id=(j,),
                    device_id_type=pl.DeviceIdType.MESH,
                )
                recv.wait_recv()

        gsum = jnp.sum(stats_ref[:, 0, :], axis=0, keepdims=True)
        stats_ref[pl.ds(my, 1), :, :] = gsum[None, :, :]

        for j in range(N_DEV):
            @pl.when(j != my)
            def _(j=j):
                pl.semaphore_signal(
                    credit_sem, inc=1,
                    device_id=(j,), device_id_type=pl.DeviceIdType.MESH,
                )

        scale_col = jnp.transpose(1.0 / gsum)
        out_ref[:, :] = e * scale_col

        pl.semaphore_wait(credit_sem, N_DEV - 1)
        pl.semaphore_wait(barrier_sem, N_DEV - 1)
        for j in range(N_DEV):
            @pl.when(j != my)
            def _(j=j):
                send = pltpu.make_async_remote_copy(
                    src_ref=local_ref,
                    dst_ref=stats_ref.at[j],
                    send_sem=send_sems.at[j],
                    recv_sem=recv_sems.at[j],
                    device_id=(j,),
                    device_id_type=pl.DeviceIdType.MESH,
                )
                send.wait_send()

    return pl.pallas_call(
        body,
        out_shape=jax.ShapeDtypeStruct((m_per, n_per), jnp.float32),
        in_specs=[pl.BlockSpec(memory_space=pltpu.VMEM)],
        out_specs=pl.BlockSpec(memory_space=pltpu.VMEM),
        scratch_shapes=[
            pltpu.VMEM((1, m_per), jnp.float32),
            pltpu.VMEM((N_DEV, 1, m_per), jnp.float32),
            pltpu.SemaphoreType.DMA((N_DEV,)),
            pltpu.SemaphoreType.DMA((N_DEV,)),
            pltpu.SemaphoreType.REGULAR,
        ],
        compiler_params=pltpu.CompilerParams(collective_id=0),
    )(x)


# device time: 16857 ns/iter; 1.0323x vs baseline; 1.0323x over previous
import jax
import jax.numpy as jnp
from jax import lax
from jax.experimental import pallas as pl
from jax.experimental.pallas import tpu as pltpu

N_DEV = 16


def kernel(x):
    m_per, n_per = x.shape

    def body(x_ref, out_ref, local_ref, stats_ref, send_sems, recv_sems):
        my = lax.axis_index("i")

        barrier_sem = pltpu.get_barrier_semaphore()
        for j in range(N_DEV):
            pl.semaphore_signal(
                barrier_sem, inc=1,
                device_id=(j,), device_id_type=pl.DeviceIdType.MESH,
            )

        xv = x_ref[:, :]
        e = jnp.exp(xv)
        s = jnp.sum(e, axis=1, keepdims=True)

        local_ref[:, :] = jnp.transpose(s)

        pl.semaphore_wait(barrier_sem, N_DEV)

        for j in range(N_DEV):
            rdma = pltpu.make_async_remote_copy(
                src_ref=local_ref,
                dst_ref=stats_ref.at[my],
                send_sem=send_sems.at[j],
                recv_sem=recv_sems.at[my],
                device_id=(j,),
                device_id_type=pl.DeviceIdType.MESH,
            )
            rdma.start()

        for j in range(N_DEV):
            recv = pltpu.make_async_remote_copy(
                src_ref=local_ref,
                dst_ref=stats_ref.at[j],
                send_sem=send_sems.at[j],
                recv_sem=recv_sems.at[j],
                device_id=(j,),
                device_id_type=pl.DeviceIdType.MESH,
            )
            recv.wait_recv()

        gsum = jnp.sum(stats_ref[:, 0, :], axis=0, keepdims=True)
        scale_col = jnp.transpose(1.0 / gsum)
        out_ref[:, :] = e * scale_col

        for j in range(N_DEV):
            send = pltpu.make_async_remote_copy(
                src_ref=local_ref,
                dst_ref=stats_ref.at[j],
                send_sem=send_sems.at[j],
                recv_sem=recv_sems.at[j],
                device_id=(j,),
                device_id_type=pl.DeviceIdType.MESH,
            )
            send.wait_send()

    return pl.pallas_call(
        body,
        out_shape=jax.ShapeDtypeStruct((m_per, n_per), jnp.float32),
        in_specs=[pl.BlockSpec(memory_space=pltpu.VMEM)],
        out_specs=pl.BlockSpec(memory_space=pltpu.VMEM),
        scratch_shapes=[
            pltpu.VMEM((1, m_per), jnp.float32),
            pltpu.VMEM((N_DEV, 1, m_per), jnp.float32),
            pltpu.SemaphoreType.DMA((N_DEV,)),
            pltpu.SemaphoreType.DMA((N_DEV,)),
        ],
        compiler_params=pltpu.CompilerParams(collective_id=0),
    )(x)
